# baseline (speedup 1.0000x reference)
import functools

import jax
import jax.numpy as jnp
from jax import lax
from jax.experimental import pallas as pl
from jax.experimental.pallas import tpu as pltpu
from jax.experimental.pallas import tpu_sc as plsc

_BLK = 8192


def _sc_gather(table2, ids2):
    B = ids2.shape[0]
    H = table2.shape[1]
    info = plsc.get_sparse_core_info()
    nc = info.num_cores
    b_per_w = 8
    n_workers = B // b_per_w
    mesh = plsc.VectorSubcoreMesh(core_axis_name="c", subcore_axis_name="s")

    @functools.partial(
        pl.kernel,
        mesh=mesh,
        out_type=jax.ShapeDtypeStruct((B, H), jnp.float32),
        scratch_types=[
            pltpu.VMEM((b_per_w,), jnp.int32),
            pltpu.VMEM((b_per_w, H), jnp.float32),
            pltpu.SemaphoreType.DMA,
        ],
    )
    def gather_k(table_hbm, idx_hbm, out_hbm, idx_v, rows_v, sem):
        wid = lax.axis_index("s") * nc + lax.axis_index("c")

        @pl.when(wid < n_workers)
        def _():
            base = wid * b_per_w
            pltpu.sync_copy(idx_hbm.at[pl.ds(base, b_per_w)], idx_v)
            pltpu.async_copy(table_hbm.at[idx_v], rows_v, sem).wait()
            pltpu.sync_copy(rows_v, out_hbm.at[pl.ds(base, b_per_w)])

    return gather_k(table2, ids2)


def _lnmm_body(x2_ref, par_ref, g_ref, bt_ref, w_ref, b_ref, eb_ref, o_ref):
    x2 = x2_ref[...]
    H = x2.shape[1] // 2
    par = par_ref[...]
    x = jnp.where(par == 0, x2[:, :H], x2[:, H:])
    mu = jnp.mean(x, axis=-1, keepdims=True)
    var = jnp.mean((x - mu) * (x - mu), axis=-1, keepdims=True)
    xn = (x - mu) * lax.rsqrt(var + 1e-5)
    xn = xn * g_ref[...] + bt_ref[...]
    acc = lax.dot_general(
        xn, w_ref[...], (((1,), (1,)), ((), ())),
        preferred_element_type=jnp.float32,
    )
    o_ref[...] = acc + b_ref[...] + eb_ref[...]


def kernel(input_ids, emb_table, ln_gamma, ln_beta, lin_W, lin_b, extra_bias):
    B, S = input_ids.shape
    V, H = emb_table.shape
    ids = input_ids.reshape(B * S).astype(jnp.int32)
    table2 = emb_table.reshape(V // 2, 2 * H)
    ids2 = ids // 2
    parity = (ids % 2).reshape(B * S, 1)

    embeds2 = _sc_gather(table2, ids2)

    nblk = -(-V // _BLK)
    logits = pl.pallas_call(
        _lnmm_body,
        grid=(nblk,),
        in_specs=[
            pl.BlockSpec((B * S, 2 * H), lambda j: (0, 0)),
            pl.BlockSpec((B * S, 1), lambda j: (0, 0)),
            pl.BlockSpec((1, H), lambda j: (0, 0)),
            pl.BlockSpec((1, H), lambda j: (0, 0)),
            pl.BlockSpec((_BLK, H), lambda j: (j, 0)),
            pl.BlockSpec((1, _BLK), lambda j: (0, j)),
            pl.BlockSpec((1, _BLK), lambda j: (0, j)),
        ],
        out_specs=pl.BlockSpec((B * S, _BLK), lambda j: (0, j)),
        out_shape=jax.ShapeDtypeStruct((B * S, V), jnp.float32),
        compiler_params=pltpu.CompilerParams(
            dimension_semantics=("arbitrary",),
        ),
    )(
        embeds2,
        parity,
        ln_gamma.reshape(1, H),
        ln_beta.reshape(1, H),
        lin_W,
        lin_b.reshape(1, V),
        extra_bias.reshape(1, V),
    )
    return logits.reshape(B, S, V)


# speedup vs baseline: 1.2829x; 1.2829x over previous
import functools

import jax
import jax.numpy as jnp
from jax import lax
from jax.experimental import pallas as pl
from jax.experimental.pallas import tpu as pltpu

_BLK = 8192


def _fused_body(ids_ref, emb_hbm, g_ref, bt_ref, w_ref, b_ref, eb_ref,
                o_ref, x_ref, sems):
    j = pl.program_id(0)
    B = x_ref.shape[0]

    @pl.when(j == 0)
    def _():
        for k in range(B):
            pltpu.make_async_copy(
                emb_hbm.at[ids_ref[k]], x_ref.at[k], sems.at[k]
            ).start()
        for k in range(B):
            pltpu.make_async_copy(
                emb_hbm.at[ids_ref[k]], x_ref.at[k], sems.at[k]
            ).wait()

    x = x_ref[...]
    mu = jnp.mean(x, axis=-1, keepdims=True)
    var = jnp.mean((x - mu) * (x - mu), axis=-1, keepdims=True)
    xn = (x - mu) * lax.rsqrt(var + 1e-5)
    xn = xn * g_ref[...] + bt_ref[...]
    acc = lax.dot_general(
        xn, w_ref[...], (((1,), (1,)), ((), ())),
        preferred_element_type=jnp.float32,
    )
    o_ref[...] = acc + (b_ref[...] + eb_ref[...])[None, :]


def kernel(input_ids, emb_table, ln_gamma, ln_beta, lin_W, lin_b, extra_bias):
    B, S = input_ids.shape
    V, H = emb_table.shape
    ids = input_ids.reshape(B * S).astype(jnp.int32)

    nblk = -(-V // _BLK)
    grid_spec = pltpu.PrefetchScalarGridSpec(
        num_scalar_prefetch=1,
        grid=(nblk,),
        in_specs=[
            pl.BlockSpec(memory_space=pl.ANY),
            pl.BlockSpec((1, H), lambda j, ids: (0, 0)),
            pl.BlockSpec((1, H), lambda j, ids: (0, 0)),
            pl.BlockSpec((_BLK, H), lambda j, ids: (j, 0)),
            pl.BlockSpec((_BLK,), lambda j, ids: (j,)),
            pl.BlockSpec((_BLK,), lambda j, ids: (j,)),
        ],
        out_specs=pl.BlockSpec((B * S, _BLK), lambda j, ids: (0, j)),
        scratch_shapes=[
            pltpu.VMEM((B * S, H), jnp.float32),
            pltpu.SemaphoreType.DMA((B * S,)),
        ],
    )
    logits = pl.pallas_call(
        _fused_body,
        grid_spec=grid_spec,
        out_shape=jax.ShapeDtypeStruct((B * S, V), jnp.float32),
        compiler_params=pltpu.CompilerParams(
            dimension_semantics=("arbitrary",),
        ),
    )(
        ids,
        emb_table,
        ln_gamma.reshape(1, H),
        ln_beta.reshape(1, H),
        lin_W,
        lin_b,
        extra_bias,
    )
    return logits.reshape(B, S, V)


# speedup vs baseline: 8.2337x; 6.4183x over previous
import jax
import jax.numpy as jnp
from jax import lax
from jax.experimental import pallas as pl
from jax.experimental.pallas import tpu as pltpu

_BLK = 8192
_LANES = 128


def _fused_body(ids_ref, embT_hbm, edge_ref, g_ref, bt_ref, w_ref, b_ref,
                eb_ref, o_ref, xall_ref, xcol_ref, xn_ref, sems):
    j = pl.program_id(0)
    B = xcol_ref.shape[1]
    V = embT_hbm.shape[1]
    n_full = V // _LANES

    @pl.when(j == 0)
    def _():
        for k in range(B):
            t = jnp.minimum(ids_ref[k] // _LANES, n_full - 1)
            base = pl.multiple_of(t * _LANES, _LANES)
            pltpu.make_async_copy(
                embT_hbm.at[:, pl.ds(base, _LANES)],
                xall_ref.at[:, pl.ds(k * _LANES, _LANES)],
                sems.at[k],
            ).start()
        for k in range(B):
            pltpu.make_async_copy(
                embT_hbm.at[:, pl.ds(0, _LANES)],
                xall_ref.at[:, pl.ds(k * _LANES, _LANES)],
                sems.at[k],
            ).wait()
        lane = lax.broadcasted_iota(jnp.int32, (1, _LANES), 1)
        for k in range(B):
            is_edge = ids_ref[k] // _LANES >= n_full
            off = ids_ref[k] % _LANES
            tile = jnp.where(
                is_edge, edge_ref[...],
                xall_ref[:, k * _LANES:(k + 1) * _LANES],
            )
            sel = jnp.where(lane == off, tile, 0.0)
            xcol_ref[:, k:k + 1] = jnp.sum(sel, axis=1, keepdims=True)
        x = jnp.transpose(xcol_ref[...])
        mu = jnp.mean(x, axis=-1, keepdims=True)
        var = jnp.mean((x - mu) * (x - mu), axis=-1, keepdims=True)
        xn = (x - mu) * lax.rsqrt(var + 1e-5)
        xn_ref[...] = xn * g_ref[...] + bt_ref[...]

    acc = lax.dot_general(
        xn_ref[...], w_ref[...], (((1,), (0,)), ((), ())),
        preferred_element_type=jnp.float32,
    )
    o_ref[...] = (acc + (b_ref[...] + eb_ref[...])[None, :])[:, None, :]


def kernel(input_ids, emb_table, ln_gamma, ln_beta, lin_W, lin_b, extra_bias):
    B, S = input_ids.shape
    V, H = emb_table.shape
    ids = input_ids.reshape(B * S).astype(jnp.int32)

    edge_t = V // _LANES if V % _LANES else V // _LANES - 1

    nblk = -(-V // _BLK)
    grid_spec = pltpu.PrefetchScalarGridSpec(
        num_scalar_prefetch=1,
        grid=(nblk,),
        in_specs=[
            pl.BlockSpec(memory_space=pl.ANY),
            pl.BlockSpec((H, _LANES), lambda j, ids: (0, edge_t)),
            pl.BlockSpec((1, H), lambda j, ids: (0, 0)),
            pl.BlockSpec((1, H), lambda j, ids: (0, 0)),
            pl.BlockSpec((H, _BLK), lambda j, ids: (0, j)),
            pl.BlockSpec((_BLK,), lambda j, ids: (j,)),
            pl.BlockSpec((_BLK,), lambda j, ids: (j,)),
        ],
        out_specs=pl.BlockSpec((B * S, 1, _BLK), lambda j, ids: (0, 0, j)),
        scratch_shapes=[
            pltpu.VMEM((H, B * S * _LANES), jnp.float32),
            pltpu.VMEM((H, B * S), jnp.float32),
            pltpu.VMEM((B * S, H), jnp.float32),
            pltpu.SemaphoreType.DMA((B * S,)),
        ],
    )
    logits = pl.pallas_call(
        _fused_body,
        grid_spec=grid_spec,
        out_shape=jax.ShapeDtypeStruct((B * S, 1, V), jnp.float32),
        compiler_params=pltpu.CompilerParams(
            dimension_semantics=("arbitrary",),
        ),
    )(
        ids,
        emb_table.T,
        emb_table.T,
        ln_gamma.reshape(1, H),
        ln_beta.reshape(1, H),
        lin_W.T,
        lin_b,
        extra_bias,
    )
    return logits.reshape(B, S, V)


# speedup vs baseline: 10.3604x; 1.2583x over previous
import jax
import jax.numpy as jnp
from jax import lax
from jax.experimental import pallas as pl
from jax.experimental.pallas import tpu as pltpu

_BLK = 16384
_LANES = 128


def _fused_body(ids_ref, embT_hbm, edge_ref, g_ref, bt_ref, w_ref, b_ref,
                eb_ref, o_ref, xall_ref, xcol_ref, xn_ref, sems):
    j = pl.program_id(0)
    B = xcol_ref.shape[1]
    V = embT_hbm.shape[1]
    n_full = V // _LANES

    @pl.when(j == 0)
    def _():
        for k in range(B):
            t = jnp.minimum(ids_ref[k] // _LANES, n_full - 1)
            base = pl.multiple_of(t * _LANES, _LANES)
            pltpu.make_async_copy(
                embT_hbm.at[:, pl.ds(base, _LANES)],
                xall_ref.at[:, pl.ds(k * _LANES, _LANES)],
                sems.at[k],
            ).start()
        for k in range(B):
            pltpu.make_async_copy(
                embT_hbm.at[:, pl.ds(0, _LANES)],
                xall_ref.at[:, pl.ds(k * _LANES, _LANES)],
                sems.at[k],
            ).wait()
        lane = lax.broadcasted_iota(jnp.int32, (1, _LANES), 1)
        for k in range(B):
            is_edge = ids_ref[k] // _LANES >= n_full
            off = ids_ref[k] % _LANES
            tile = jnp.where(
                is_edge, edge_ref[...],
                xall_ref[:, k * _LANES:(k + 1) * _LANES],
            )
            sel = jnp.where(lane == off, tile, 0.0)
            xcol_ref[:, k:k + 1] = jnp.sum(sel, axis=1, keepdims=True)
        x = jnp.transpose(xcol_ref[...])
        mu = jnp.mean(x, axis=-1, keepdims=True)
        var = jnp.mean((x - mu) * (x - mu), axis=-1, keepdims=True)
        xn = (x - mu) * lax.rsqrt(var + 1e-5)
        xn_ref[...] = xn * g_ref[...] + bt_ref[...]

    acc = lax.dot_general(
        xn_ref[...], w_ref[...], (((1,), (0,)), ((), ())),
        preferred_element_type=jnp.float32,
    )
    o_ref[...] = (acc + (b_ref[...] + eb_ref[...])[None, :])[:, None, :]


def kernel(input_ids, emb_table, ln_gamma, ln_beta, lin_W, lin_b, extra_bias):
    B, S = input_ids.shape
    V, H = emb_table.shape
    ids = input_ids.reshape(B * S).astype(jnp.int32)

    edge_t = V // _LANES if V % _LANES else V // _LANES - 1

    nblk = -(-V // _BLK)
    grid_spec = pltpu.PrefetchScalarGridSpec(
        num_scalar_prefetch=1,
        grid=(nblk,),
        in_specs=[
            pl.BlockSpec(memory_space=pl.ANY),
            pl.BlockSpec((H, _LANES), lambda j, ids: (0, edge_t)),
            pl.BlockSpec((1, H), lambda j, ids: (0, 0)),
            pl.BlockSpec((1, H), lambda j, ids: (0, 0)),
            pl.BlockSpec((H, _BLK), lambda j, ids: (0, j)),
            pl.BlockSpec((_BLK,), lambda j, ids: (j,)),
            pl.BlockSpec((_BLK,), lambda j, ids: (j,)),
        ],
        out_specs=pl.BlockSpec((B * S, 1, _BLK), lambda j, ids: (0, 0, j)),
        scratch_shapes=[
            pltpu.VMEM((H, B * S * _LANES), jnp.float32),
            pltpu.VMEM((H, B * S), jnp.float32),
            pltpu.VMEM((B * S, H), jnp.float32),
            pltpu.SemaphoreType.DMA((B * S,)),
        ],
    )
    logits = pl.pallas_call(
        _fused_body,
        grid_spec=grid_spec,
        out_shape=jax.ShapeDtypeStruct((B * S, 1, V), jnp.float32),
        compiler_params=pltpu.CompilerParams(
            dimension_semantics=("arbitrary",),
        ),
    )(
        ids,
        emb_table.T,
        emb_table.T,
        ln_gamma.reshape(1, H),
        ln_beta.reshape(1, H),
        lin_W.T,
        lin_b,
        extra_bias,
    )
    return logits.reshape(B, S, V)


# speedup vs baseline: 10.7978x; 1.0422x over previous
import jax
import jax.numpy as jnp
from jax import lax
from jax.experimental import pallas as pl
from jax.experimental.pallas import tpu as pltpu

_BLK = 32768
_LANES = 128


def _fused_body(ids_ref, embT_hbm, edge_ref, g_ref, bt_ref, w_ref, b_ref,
                eb_ref, o_ref, xall_ref, xcol_ref, xn_ref, sems):
    j = pl.program_id(0)
    B = xcol_ref.shape[1]
    V = embT_hbm.shape[1]
    n_full = V // _LANES

    @pl.when(j == 0)
    def _():
        for k in range(B):
            t = jnp.minimum(ids_ref[k] // _LANES, n_full - 1)
            base = pl.multiple_of(t * _LANES, _LANES)
            pltpu.make_async_copy(
                embT_hbm.at[:, pl.ds(base, _LANES)],
                xall_ref.at[:, pl.ds(k * _LANES, _LANES)],
                sems.at[k],
            ).start()
        for k in range(B):
            pltpu.make_async_copy(
                embT_hbm.at[:, pl.ds(0, _LANES)],
                xall_ref.at[:, pl.ds(k * _LANES, _LANES)],
                sems.at[k],
            ).wait()
        lane = lax.broadcasted_iota(jnp.int32, (1, _LANES), 1)
        for k in range(B):
            is_edge = ids_ref[k] // _LANES >= n_full
            off = ids_ref[k] % _LANES
            tile = jnp.where(
                is_edge, edge_ref[...],
                xall_ref[:, k * _LANES:(k + 1) * _LANES],
            )
            sel = jnp.where(lane == off, tile, 0.0)
            xcol_ref[:, k:k + 1] = jnp.sum(sel, axis=1, keepdims=True)
        x = jnp.transpose(xcol_ref[...])
        mu = jnp.mean(x, axis=-1, keepdims=True)
        var = jnp.mean((x - mu) * (x - mu), axis=-1, keepdims=True)
        xn = (x - mu) * lax.rsqrt(var + 1e-5)
        xn_ref[...] = xn * g_ref[...] + bt_ref[...]

    acc = lax.dot_general(
        xn_ref[...], w_ref[...], (((1,), (0,)), ((), ())),
        preferred_element_type=jnp.float32,
    )
    o_ref[...] = (acc + (b_ref[...] + eb_ref[...])[None, :])[:, None, :]


def kernel(input_ids, emb_table, ln_gamma, ln_beta, lin_W, lin_b, extra_bias):
    B, S = input_ids.shape
    V, H = emb_table.shape
    ids = input_ids.reshape(B * S).astype(jnp.int32)

    edge_t = V // _LANES if V % _LANES else V // _LANES - 1

    nblk = -(-V // _BLK)
    grid_spec = pltpu.PrefetchScalarGridSpec(
        num_scalar_prefetch=1,
        grid=(nblk,),
        in_specs=[
            pl.BlockSpec(memory_space=pl.ANY),
            pl.BlockSpec((H, _LANES), lambda j, ids: (0, edge_t)),
            pl.BlockSpec((1, H), lambda j, ids: (0, 0)),
            pl.BlockSpec((1, H), lambda j, ids: (0, 0)),
            pl.BlockSpec((H, _BLK), lambda j, ids: (0, j)),
            pl.BlockSpec((_BLK,), lambda j, ids: (j,)),
            pl.BlockSpec((_BLK,), lambda j, ids: (j,)),
        ],
        out_specs=pl.BlockSpec((B * S, 1, _BLK), lambda j, ids: (0, 0, j)),
        scratch_shapes=[
            pltpu.VMEM((H, B * S * _LANES), jnp.float32),
            pltpu.VMEM((H, B * S), jnp.float32),
            pltpu.VMEM((B * S, H), jnp.float32),
            pltpu.SemaphoreType.DMA((B * S,)),
        ],
    )
    logits = pl.pallas_call(
        _fused_body,
        grid_spec=grid_spec,
        out_shape=jax.ShapeDtypeStruct((B * S, 1, V), jnp.float32),
        compiler_params=pltpu.CompilerParams(
            dimension_semantics=("arbitrary",),
        ),
    )(
        ids,
        emb_table.T,
        emb_table.T,
        ln_gamma.reshape(1, H),
        ln_beta.reshape(1, H),
        lin_W.T,
        lin_b,
        extra_bias,
    )
    return logits.reshape(B, S, V)


# speedup vs baseline: 10.9063x; 1.0100x over previous
import jax
import jax.numpy as jnp
from jax import lax
from jax.experimental import pallas as pl
from jax.experimental.pallas import tpu as pltpu

_BLK = 65536
_LANES = 128


def _fused_body(ids_ref, embT_hbm, edge_ref, g_ref, bt_ref, w_ref, b_ref,
                eb_ref, o_ref, xall_ref, xcol_ref, xn_ref, sems):
    j = pl.program_id(0)
    B = xcol_ref.shape[1]
    V = embT_hbm.shape[1]
    n_full = V // _LANES

    @pl.when(j == 0)
    def _():
        for k in range(B):
            t = jnp.minimum(ids_ref[k] // _LANES, n_full - 1)
            base = pl.multiple_of(t * _LANES, _LANES)
            pltpu.make_async_copy(
                embT_hbm.at[:, pl.ds(base, _LANES)],
                xall_ref.at[:, pl.ds(k * _LANES, _LANES)],
                sems.at[k],
            ).start()
        for k in range(B):
            pltpu.make_async_copy(
                embT_hbm.at[:, pl.ds(0, _LANES)],
                xall_ref.at[:, pl.ds(k * _LANES, _LANES)],
                sems.at[k],
            ).wait()
        lane = lax.broadcasted_iota(jnp.int32, (1, _LANES), 1)
        for k in range(B):
            is_edge = ids_ref[k] // _LANES >= n_full
            off = ids_ref[k] % _LANES
            tile = jnp.where(
                is_edge, edge_ref[...],
                xall_ref[:, k * _LANES:(k + 1) * _LANES],
            )
            sel = jnp.where(lane == off, tile, 0.0)
            xcol_ref[:, k:k + 1] = jnp.sum(sel, axis=1, keepdims=True)
        x = jnp.transpose(xcol_ref[...])
        mu = jnp.mean(x, axis=-1, keepdims=True)
        var = jnp.mean((x - mu) * (x - mu), axis=-1, keepdims=True)
        xn = (x - mu) * lax.rsqrt(var + 1e-5)
        xn_ref[...] = xn * g_ref[...] + bt_ref[...]

    acc = lax.dot_general(
        xn_ref[...], w_ref[...], (((1,), (0,)), ((), ())),
        preferred_element_type=jnp.float32,
    )
    o_ref[...] = (acc + (b_ref[...] + eb_ref[...])[None, :])[:, None, :]


def kernel(input_ids, emb_table, ln_gamma, ln_beta, lin_W, lin_b, extra_bias):
    B, S = input_ids.shape
    V, H = emb_table.shape
    ids = input_ids.reshape(B * S).astype(jnp.int32)

    edge_t = V // _LANES if V % _LANES else V // _LANES - 1

    nblk = -(-V // _BLK)
    grid_spec = pltpu.PrefetchScalarGridSpec(
        num_scalar_prefetch=1,
        grid=(nblk,),
        in_specs=[
            pl.BlockSpec(memory_space=pl.ANY),
            pl.BlockSpec((H, _LANES), lambda j, ids: (0, edge_t)),
            pl.BlockSpec((1, H), lambda j, ids: (0, 0)),
            pl.BlockSpec((1, H), lambda j, ids: (0, 0)),
            pl.BlockSpec((H, _BLK), lambda j, ids: (0, j)),
            pl.BlockSpec((_BLK,), lambda j, ids: (j,)),
            pl.BlockSpec((_BLK,), lambda j, ids: (j,)),
        ],
        out_specs=pl.BlockSpec((B * S, 1, _BLK), lambda j, ids: (0, 0, j)),
        scratch_shapes=[
            pltpu.VMEM((H, B * S * _LANES), jnp.float32),
            pltpu.VMEM((H, B * S), jnp.float32),
            pltpu.VMEM((B * S, H), jnp.float32),
            pltpu.SemaphoreType.DMA((B * S,)),
        ],
    )
    logits = pl.pallas_call(
        _fused_body,
        grid_spec=grid_spec,
        out_shape=jax.ShapeDtypeStruct((B * S, 1, V), jnp.float32),
        compiler_params=pltpu.CompilerParams(
            dimension_semantics=("arbitrary",),
        ),
    )(
        ids,
        emb_table.T,
        emb_table.T,
        ln_gamma.reshape(1, H),
        ln_beta.reshape(1, H),
        lin_W.T,
        lin_b,
        extra_bias,
    )
    return logits.reshape(B, S, V)


# speedup vs baseline: 10.9479x; 1.0038x over previous
import jax
import jax.numpy as jnp
from jax import lax
from jax.experimental import pallas as pl
from jax.experimental.pallas import tpu as pltpu

_BLK = 73728
_LANES = 128


def _fused_body(ids_ref, embT_hbm, edge_ref, g_ref, bt_ref, w_ref, b_ref,
                eb_ref, o_ref, xall_ref, xcol_ref, xn_ref, sems):
    j = pl.program_id(0)
    B = xcol_ref.shape[1]
    V = embT_hbm.shape[1]
    n_full = V // _LANES

    @pl.when(j == 0)
    def _():
        for k in range(B):
            t = jnp.minimum(ids_ref[k] // _LANES, n_full - 1)
            base = pl.multiple_of(t * _LANES, _LANES)
            pltpu.make_async_copy(
                embT_hbm.at[:, pl.ds(base, _LANES)],
                xall_ref.at[:, pl.ds(k * _LANES, _LANES)],
                sems.at[k],
            ).start()
        for k in range(B):
            pltpu.make_async_copy(
                embT_hbm.at[:, pl.ds(0, _LANES)],
                xall_ref.at[:, pl.ds(k * _LANES, _LANES)],
                sems.at[k],
            ).wait()
        lane = lax.broadcasted_iota(jnp.int32, (1, _LANES), 1)
        for k in range(B):
            is_edge = ids_ref[k] // _LANES >= n_full
            off = ids_ref[k] % _LANES
            tile = jnp.where(
                is_edge, edge_ref[...],
                xall_ref[:, k * _LANES:(k + 1) * _LANES],
            )
            sel = jnp.where(lane == off, tile, 0.0)
            xcol_ref[:, k:k + 1] = jnp.sum(sel, axis=1, keepdims=True)
        x = jnp.transpose(xcol_ref[...])
        mu = jnp.mean(x, axis=-1, keepdims=True)
        var = jnp.mean((x - mu) * (x - mu), axis=-1, keepdims=True)
        xn = (x - mu) * lax.rsqrt(var + 1e-5)
        xn_ref[...] = xn * g_ref[...] + bt_ref[...]

    acc = lax.dot_general(
        xn_ref[...], w_ref[...], (((1,), (0,)), ((), ())),
        preferred_element_type=jnp.float32,
    )
    o_ref[...] = (acc + (b_ref[...] + eb_ref[...])[None, :])[:, None, :]


def kernel(input_ids, emb_table, ln_gamma, ln_beta, lin_W, lin_b, extra_bias):
    B, S = input_ids.shape
    V, H = emb_table.shape
    ids = input_ids.reshape(B * S).astype(jnp.int32)

    edge_t = V // _LANES if V % _LANES else V // _LANES - 1

    nblk = -(-V // _BLK)
    grid_spec = pltpu.PrefetchScalarGridSpec(
        num_scalar_prefetch=1,
        grid=(nblk,),
        in_specs=[
            pl.BlockSpec(memory_space=pl.ANY),
            pl.BlockSpec((H, _LANES), lambda j, ids: (0, edge_t)),
            pl.BlockSpec((1, H), lambda j, ids: (0, 0)),
            pl.BlockSpec((1, H), lambda j, ids: (0, 0)),
            pl.BlockSpec((H, _BLK), lambda j, ids: (0, j)),
            pl.BlockSpec((_BLK,), lambda j, ids: (j,)),
            pl.BlockSpec((_BLK,), lambda j, ids: (j,)),
        ],
        out_specs=pl.BlockSpec((B * S, 1, _BLK), lambda j, ids: (0, 0, j)),
        scratch_shapes=[
            pltpu.VMEM((H, B * S * _LANES), jnp.float32),
            pltpu.VMEM((H, B * S), jnp.float32),
            pltpu.VMEM((B * S, H), jnp.float32),
            pltpu.SemaphoreType.DMA((B * S,)),
        ],
    )
    logits = pl.pallas_call(
        _fused_body,
        grid_spec=grid_spec,
        out_shape=jax.ShapeDtypeStruct((B * S, 1, V), jnp.float32),
        compiler_params=pltpu.CompilerParams(
            dimension_semantics=("arbitrary",),
        ),
    )(
        ids,
        emb_table.T,
        emb_table.T,
        ln_gamma.reshape(1, H),
        ln_beta.reshape(1, H),
        lin_W.T,
        lin_b,
        extra_bias,
    )
    return logits.reshape(B, S, V)
